# baseline (device time: 37855 ns/iter reference)
import jax
import jax.numpy as jnp
from jax import lax
from jax.experimental import pallas as pl
from jax.experimental.pallas import tpu as pltpu

Z = 4


def kernel(Q, K, V):
    b, q, h, d = Q.shape
    scale = d ** -0.5

    def body(q_ref, k_ref, v_ref, out_ref, send_buf, comm_ref, send_sems, recv_sems):
        my_x = lax.axis_index("x")
        my_y = lax.axis_index("y")
        my_z = lax.axis_index("z")

        qv = q_ref[:]
        kv = k_ref[:]
        s = jnp.sum(qv * kv, axis=3) * scale
        p = jnp.exp(s)
        l = jnp.sum(p, axis=1)
        o = jnp.sum(p[:, :, :, None] * v_ref[:], axis=1)

        l_row = jnp.pad(l[:, None, :], ((0, 0), (0, 0), (0, d - h)))
        send_buf[:] = jnp.concatenate([o, l_row], axis=1)

        barrier_sem = pltpu.get_barrier_semaphore()
        for off in range(1, Z):
            pl.semaphore_signal(
                barrier_sem,
                inc=1,
                device_id=(my_x, my_y, (my_z + off) % Z),
                device_id_type=pl.DeviceIdType.MESH,
            )
        pl.semaphore_wait(barrier_sem, Z - 1)

        rdmas = []
        for off in range(1, Z):
            rdma = pltpu.make_async_remote_copy(
                src_ref=send_buf,
                dst_ref=comm_ref.at[off - 1],
                send_sem=send_sems.at[off - 1],
                recv_sem=recv_sems.at[off - 1],
                device_id=(my_x, my_y, (my_z + off) % Z),
                device_id_type=pl.DeviceIdType.MESH,
            )
            rdma.start()
            rdmas.append(rdma)

        for rdma in rdmas:
            rdma.wait_recv()

        total = send_buf[:] + comm_ref[0] + comm_ref[1] + comm_ref[2]
        o_sum = total[:, :h, :]
        l_sum = total[:, h, :h]
        out_ref[:] = (o_sum / l_sum[:, :, None])[:, None, :, :]

        for rdma in rdmas:
            rdma.wait_send()

    out_shape = jax.ShapeDtypeStruct((b, q, h, d), jnp.float32)
    return pl.pallas_call(
        body,
        out_shape=out_shape,
        in_specs=[pl.BlockSpec(memory_space=pltpu.VMEM)] * 3,
        out_specs=pl.BlockSpec(memory_space=pltpu.VMEM),
        scratch_shapes=[
            pltpu.VMEM((b, h + 1, d), jnp.float32),
            pltpu.VMEM((Z - 1, b, h + 1, d), jnp.float32),
            pltpu.SemaphoreType.DMA((Z - 1,)),
            pltpu.SemaphoreType.DMA((Z - 1,)),
        ],
        compiler_params=pltpu.CompilerParams(collective_id=0),
    )(Q, K, V)


# device time: 36740 ns/iter; 1.0303x vs baseline; 1.0303x over previous
import jax
import jax.numpy as jnp
from jax import lax
from jax.experimental import pallas as pl
from jax.experimental.pallas import tpu as pltpu

Z = 4


def kernel(Q, K, V):
    b, q, h, d = Q.shape
    scale = d ** -0.5

    def body(q_ref, k_ref, v_ref, out_ref, send_buf, comm_ref, send_sems, recv_sems):
        my_x = lax.axis_index("x")
        my_y = lax.axis_index("y")
        my_z = lax.axis_index("z")

        qv = q_ref[:] * scale
        kv = k_ref[:]
        p = jnp.exp(jnp.sum(qv * kv, axis=3, keepdims=True))
        l = jnp.sum(p, axis=1)[:, :, 0]
        o = jnp.sum(p * v_ref[:], axis=1)

        l_row = jnp.pad(l[:, None, :], ((0, 0), (0, 0), (0, d - h)))
        send_buf[:] = jnp.concatenate([o, l_row], axis=1)

        barrier_sem = pltpu.get_barrier_semaphore()
        for off in range(1, Z):
            pl.semaphore_signal(
                barrier_sem,
                inc=1,
                device_id=(my_x, my_y, (my_z + off) % Z),
                device_id_type=pl.DeviceIdType.MESH,
            )
        pl.semaphore_wait(barrier_sem, Z - 1)

        rdmas = []
        for off in range(1, Z):
            rdma = pltpu.make_async_remote_copy(
                src_ref=send_buf,
                dst_ref=comm_ref.at[off - 1],
                send_sem=send_sems.at[off - 1],
                recv_sem=recv_sems.at[off - 1],
                device_id=(my_x, my_y, (my_z + off) % Z),
                device_id_type=pl.DeviceIdType.MESH,
            )
            rdma.start()
            rdmas.append(rdma)

        for rdma in rdmas:
            rdma.wait_recv()

        total = send_buf[:] + comm_ref[0] + comm_ref[1] + comm_ref[2]
        o_sum = total[:, :h, :]
        l_sum = total[:, h, :h]
        out_ref[:] = (o_sum / l_sum[:, :, None])[:, None, :, :]

        for rdma in rdmas:
            rdma.wait_send()

    out_shape = jax.ShapeDtypeStruct((b, q, h, d), jnp.float32)
    return pl.pallas_call(
        body,
        out_shape=out_shape,
        in_specs=[pl.BlockSpec(memory_space=pltpu.VMEM)] * 3,
        out_specs=pl.BlockSpec(memory_space=pltpu.VMEM),
        scratch_shapes=[
            pltpu.VMEM((b, h + 1, d), jnp.float32),
            pltpu.VMEM((Z - 1, b, h + 1, d), jnp.float32),
            pltpu.SemaphoreType.DMA((Z - 1,)),
            pltpu.SemaphoreType.DMA((Z - 1,)),
        ],
        compiler_params=pltpu.CompilerParams(collective_id=0),
    )(Q, K, V)


# device time: 17394 ns/iter; 2.1763x vs baseline; 2.1122x over previous
import jax
import jax.numpy as jnp
from jax import lax
from jax.experimental import pallas as pl
from jax.experimental.pallas import tpu as pltpu

Z = 4


def kernel(Q, K, V):
    b, q, h, d = Q.shape
    scale = d ** -0.5

    Kt = jnp.transpose(K, (0, 2, 3, 1))
    Vt = jnp.transpose(V, (0, 2, 3, 1))
    Qt = jnp.transpose(Q * scale, (0, 2, 3, 1))

    def body(q_ref, k_ref, v_ref, out_ref, send_buf, comm_ref, send_sems, recv_sems):
        my_x = lax.axis_index("x")
        my_y = lax.axis_index("y")
        my_z = lax.axis_index("z")

        qt = q_ref[:]
        kt = k_ref[:]
        p = jnp.exp(jnp.sum(qt * kt, axis=2, keepdims=True))
        l = jnp.sum(p, axis=3)[:, :, 0]
        o = jnp.sum(p * v_ref[:], axis=3)

        l_row = jnp.pad(l[:, None, :], ((0, 0), (0, 0), (0, d - h)))
        send_buf[:] = jnp.concatenate([o, l_row], axis=1)

        barrier_sem = pltpu.get_barrier_semaphore()
        for off in range(1, Z):
            pl.semaphore_signal(
                barrier_sem,
                inc=1,
                device_id=(my_x, my_y, (my_z + off) % Z),
                device_id_type=pl.DeviceIdType.MESH,
            )
        pl.semaphore_wait(barrier_sem, Z - 1)

        rdmas = []
        for off in range(1, Z):
            rdma = pltpu.make_async_remote_copy(
                src_ref=send_buf,
                dst_ref=comm_ref.at[off - 1],
                send_sem=send_sems.at[off - 1],
                recv_sem=recv_sems.at[off - 1],
                device_id=(my_x, my_y, (my_z + off) % Z),
                device_id_type=pl.DeviceIdType.MESH,
            )
            rdma.start()
            rdmas.append(rdma)

        for rdma in rdmas:
            rdma.wait_recv()

        total = send_buf[:] + comm_ref[0] + comm_ref[1] + comm_ref[2]
        o_sum = total[:, :h, :]
        l_sum = total[:, h, :h]
        out_ref[:] = (o_sum / l_sum[:, :, None])[:, None, :, :]

        for rdma in rdmas:
            rdma.wait_send()

    out_shape = jax.ShapeDtypeStruct((b, q, h, d), jnp.float32)
    return pl.pallas_call(
        body,
        out_shape=out_shape,
        in_specs=[pl.BlockSpec(memory_space=pltpu.VMEM)] * 3,
        out_specs=pl.BlockSpec(memory_space=pltpu.VMEM),
        scratch_shapes=[
            pltpu.VMEM((b, h + 1, d), jnp.float32),
            pltpu.VMEM((Z - 1, b, h + 1, d), jnp.float32),
            pltpu.SemaphoreType.DMA((Z - 1,)),
            pltpu.SemaphoreType.DMA((Z - 1,)),
        ],
        compiler_params=pltpu.CompilerParams(collective_id=0),
    )(Qt, Kt, Vt)
